# baseline (speedup 1.0000x reference)
import functools

import jax
import jax.numpy as jnp
from jax import lax
from jax.experimental import pallas as pl
from jax.experimental.pallas import tpu as pltpu
from jax.experimental.pallas import tpu_sc as plsc

_NC = 2
_NS = 16
_NW = _NC * _NS

_SCH = 4
_BT = 128
_L = 16


def _embed_call(seq, d, batch):
  n_chunks = seq // _SCH
  dt = d // 8
  mesh = plsc.VectorSubcoreMesh(core_axis_name="c", subcore_axis_name="s")

  @functools.partial(
      pl.kernel,
      mesh=mesh,
      out_type=jax.ShapeDtypeStruct((seq, dt, batch // _BT, 8, _BT),
                                    jnp.float32),
      compiler_params=pltpu.CompilerParams(use_tc_tiling_on_sc=False,
                                           needs_layout_passes=False),
      scratch_types=[
          [pltpu.VMEM((_SCH, _BT), jnp.int32)] * 2,
          [pltpu.VMEM((_SCH * _BT, d), jnp.float32)] * 2,
          [pltpu.VMEM((_SCH, dt, 1, 8, _BT), jnp.float32)] * 2,
          [pltpu.SemaphoreType.DMA] * 2,
          [pltpu.SemaphoreType.DMA] * 2,
      ],
  )
  def body(table_hbm, idx_hbm, out_hbm, idx_b, rows_g, rows_t, gsem, wsem):
    wid = lax.axis_index("s") * _NC + lax.axis_index("c")
    b0 = wid * _BT
    iota = lax.iota(jnp.int32, _L)

    def stage(c, b):
      pltpu.sync_copy(idx_hbm.at[pl.ds(c * _SCH, _SCH), pl.ds(b0, _BT)],
                      idx_b[b])

    def fire(c, b):
      for so in range(_SCH):
        pltpu.async_copy(table_hbm.at[idx_b[b].at[so]],
                         rows_g[b].at[pl.ds(so * _BT, _BT)], gsem[b])

    def drain(c, b):
      for so in range(_SCH):
        pltpu.make_async_copy(table_hbm.at[idx_b[b].at[so]],
                              rows_g[b].at[pl.ds(so * _BT, _BT)],
                              gsem[b]).wait()

    def transpose(b):
      g, t = rows_g[b], rows_t[b]

      @pl.loop(0, _SCH)
      def _t(so):
        so_base = so * _BT
        for bq in range(_BT // _L):
          ridx = iota + (so_base + bq * _L)
          for tr in range(dt):
            for rr in range(8):
              dd = tr * 8 + rr
              vals = plsc.load_gather(
                  g, [ridx, jnp.full((_L,), dd, jnp.int32)])
              t[so, tr, 0, rr, pl.ds(bq * _L, _L)] = vals

    def out_slice(c):
      return out_hbm.at[pl.ds(c * _SCH, _SCH), :, pl.ds(wid, 1), :, :]

    def wb_fire(c, b):
      pltpu.async_copy(rows_t[b], out_slice(c), wsem[b])

    def wb_wait(c, b):
      pltpu.make_async_copy(rows_t[b], out_slice(c), wsem[b]).wait()

    stage(0, 0)
    fire(0, 0)
    stage(1, 1)
    fire(1, 1)

    @pl.loop(0, n_chunks // 2)
    def _pair(h):
      for r in range(2):
        c = 2 * h + r
        drain(c, r)

        @pl.when(c >= 2)
        def _():
          wb_wait(c - 2, r)

        transpose(r)
        wb_fire(c, r)

        @pl.when(c <= n_chunks - 3)
        def _():
          stage(c + 2, r)
          fire(c + 2, r)

    wb_wait(n_chunks - 2, 0)
    wb_wait(n_chunks - 1, 1)

  return body


def kernel(indices, table):
  batch, seq = indices.shape
  vocab1, d = table.shape

  idx_sb = (indices.astype(jnp.int32) + 1).T

  p = _embed_call(seq, d, batch)(table, idx_sb)
  return p.transpose(2, 4, 0, 1, 3).reshape(batch, seq, d)


# speedup vs baseline: 1.9942x; 1.9942x over previous
import functools

import jax
import jax.numpy as jnp
from jax import lax
from jax.experimental import pallas as pl
from jax.experimental.pallas import tpu as pltpu
from jax.experimental.pallas import tpu_sc as plsc

_NC = 2
_NS = 16
_NW = _NC * _NS

_SCH = 4
_BT = 128
_L = 16


def _embed_call(seq, d, batch):
  n_chunks = seq // _SCH
  dt = d // 8
  mesh = plsc.VectorSubcoreMesh(core_axis_name="c", subcore_axis_name="s")

  @functools.partial(
      pl.kernel,
      mesh=mesh,
      out_type=jax.ShapeDtypeStruct((seq, dt, batch // _BT, 8, _BT),
                                    jnp.float32),
      compiler_params=pltpu.CompilerParams(use_tc_tiling_on_sc=False,
                                           needs_layout_passes=False),
      scratch_types=[
          [pltpu.VMEM((_SCH, _BT), jnp.int32)] * 2,
          [pltpu.VMEM((_SCH * _BT, d), jnp.float32)] * 2,
          [pltpu.VMEM((_SCH, dt, 1, 8, _BT), jnp.float32)] * 2,
          [pltpu.SemaphoreType.DMA] * 2,
          [pltpu.SemaphoreType.DMA] * 2,
      ],
  )
  def body(table_hbm, idx_hbm, out_hbm, idx_b, rows_g, rows_t, gsem, wsem):
    wid = lax.axis_index("s") * _NC + lax.axis_index("c")
    b0 = wid * _BT
    iota = lax.iota(jnp.int32, _L)

    def stage(c, b):
      pltpu.sync_copy(idx_hbm.at[pl.ds(c * _SCH, _SCH), pl.ds(b0, _BT)],
                      idx_b[b])

    def fire(c, b):
      for so in range(_SCH):
        pltpu.async_copy(table_hbm.at[idx_b[b].at[so]],
                         rows_g[b].at[pl.ds(so * _BT, _BT)], gsem[b])

    def drain(c, b):
      for so in range(_SCH):
        pltpu.make_async_copy(table_hbm.at[idx_b[b].at[so]],
                              rows_g[b].at[pl.ds(so * _BT, _BT)],
                              gsem[b]).wait()

    dvecs = [jnp.full((_L,), dd, jnp.int32) for dd in range(d)]

    def transpose(b):
      g, t = rows_g[b], rows_t[b]

      @functools.partial(plsc.parallel_loop, 0, _SCH, unroll=2)
      def _t(so):
        so_base = so * _BT
        for bq in range(_BT // _L):
          ridx = iota + (so_base + bq * _L)
          for tr in range(dt):
            for rr in range(8):
              dd = tr * 8 + rr
              vals = plsc.load_gather(g, [ridx, dvecs[dd]])
              t[so, tr, 0, rr, pl.ds(bq * _L, _L)] = vals

    def out_slice(c):
      return out_hbm.at[pl.ds(c * _SCH, _SCH), :, pl.ds(wid, 1), :, :]

    def wb_fire(c, b):
      pltpu.async_copy(rows_t[b], out_slice(c), wsem[b])

    def wb_wait(c, b):
      pltpu.make_async_copy(rows_t[b], out_slice(c), wsem[b]).wait()

    stage(0, 0)
    fire(0, 0)
    stage(1, 1)
    fire(1, 1)

    @pl.loop(0, n_chunks // 2)
    def _pair(h):
      for r in range(2):
        c = 2 * h + r
        drain(c, r)

        @pl.when(c >= 2)
        def _():
          wb_wait(c - 2, r)

        transpose(r)
        wb_fire(c, r)

        @pl.when(c <= n_chunks - 3)
        def _():
          stage(c + 2, r)
          fire(c + 2, r)

    wb_wait(n_chunks - 2, 0)
    wb_wait(n_chunks - 1, 1)

  return body


def kernel(indices, table):
  batch, seq = indices.shape
  vocab1, d = table.shape

  idx_sb = (indices.astype(jnp.int32) + 1).T

  p = _embed_call(seq, d, batch)(table, idx_sb)
  return p.transpose(2, 4, 0, 1, 3).reshape(batch, seq, d)


# speedup vs baseline: 2.1092x; 1.0577x over previous
import functools

import jax
import jax.numpy as jnp
from jax import lax
from jax.experimental import pallas as pl
from jax.experimental.pallas import tpu as pltpu
from jax.experimental.pallas import tpu_sc as plsc

_NC = 2
_NS = 16
_NW = _NC * _NS

_SCH = 4
_BT = 128
_L = 16
_HW = 16


def _embed_call(seq, d, batch):
  n_chunks = seq // _SCH
  dt = d // 8
  rpb = _BT * (d // _HW)
  mesh = plsc.VectorSubcoreMesh(core_axis_name="c", subcore_axis_name="s")

  @functools.partial(
      pl.kernel,
      mesh=mesh,
      out_type=jax.ShapeDtypeStruct((seq, dt, batch // _BT, 8, _BT),
                                    jnp.float32),
      compiler_params=pltpu.CompilerParams(use_tc_tiling_on_sc=False,
                                           needs_layout_passes=False),
      scratch_types=[
          [pltpu.VMEM((_SCH, rpb), jnp.int32)] * 2,
          [pltpu.VMEM((_SCH * rpb, _HW), jnp.float32)] * 2,
          [pltpu.VMEM((_SCH, dt, 1, 8, _BT), jnp.float32)] * 2,
          [pltpu.SemaphoreType.DMA] * 2,
          [pltpu.SemaphoreType.DMA] * 2,
      ],
  )
  def body(table_hbm, idx_hbm, out_hbm, idx_b, rows_g, rows_t, gsem, wsem):
    wid = lax.axis_index("s") * _NC + lax.axis_index("c")
    r0 = wid * rpb
    iota = lax.iota(jnp.int32, _L)
    iota2 = iota * 2

    def stage(c, b):
      pltpu.sync_copy(idx_hbm.at[pl.ds(c * _SCH, _SCH), pl.ds(r0, rpb)],
                      idx_b[b])

    def fire(c, b):
      for so in range(_SCH):
        pltpu.async_copy(table_hbm.at[idx_b[b].at[so]],
                         rows_g[b].at[pl.ds(so * rpb, rpb)], gsem[b])

    def drain(c, b):
      for so in range(_SCH):
        pltpu.make_async_copy(table_hbm.at[idx_b[b].at[so]],
                              rows_g[b].at[pl.ds(so * rpb, rpb)],
                              gsem[b]).wait()

    dvecs = [jnp.full((_L,), dd, jnp.int32) for dd in range(_HW)]

    def transpose(b):
      g, t = rows_g[b], rows_t[b]

      @functools.partial(plsc.parallel_loop, 0, _SCH, unroll=2)
      def _t(so):
        so_base = so * rpb
        for bq in range(_BT // _L):
          ridx0 = iota2 + (so_base + bq * _L * 2)
          ridx1 = ridx0 + 1
          for tr in range(dt):
            for rr in range(8):
              dd = tr * 8 + rr
              ridx = ridx0 if dd < _HW else ridx1
              vals = plsc.load_gather(g, [ridx, dvecs[dd % _HW]])
              t[so, tr, 0, rr, pl.ds(bq * _L, _L)] = vals

    def out_slice(c):
      return out_hbm.at[pl.ds(c * _SCH, _SCH), :, pl.ds(wid, 1), :, :]

    def wb_fire(c, b):
      pltpu.async_copy(rows_t[b], out_slice(c), wsem[b])

    def wb_wait(c, b):
      pltpu.make_async_copy(rows_t[b], out_slice(c), wsem[b]).wait()

    stage(0, 0)
    fire(0, 0)
    stage(1, 1)
    fire(1, 1)

    @pl.loop(0, n_chunks // 2)
    def _pair(h):
      for r in range(2):
        c = 2 * h + r
        drain(c, r)

        @pl.when(c >= 2)
        def _():
          wb_wait(c - 2, r)

        transpose(r)
        wb_fire(c, r)

        @pl.when(c <= n_chunks - 3)
        def _():
          stage(c + 2, r)
          fire(c + 2, r)

    wb_wait(n_chunks - 2, 0)
    wb_wait(n_chunks - 1, 1)

  return body


def kernel(indices, table):
  batch, seq = indices.shape
  vocab1, d = table.shape
  rpr = d // _HW

  idx_sb = (indices.astype(jnp.int32) + 1).T
  idx16 = (idx_sb[:, :, None] * rpr
           + jnp.arange(rpr, dtype=jnp.int32)).reshape(seq, batch * rpr)

  table16 = table.reshape(vocab1 * rpr, _HW)

  p = _embed_call(seq, d, batch)(table16, idx16)
  return p.transpose(2, 4, 0, 1, 3).reshape(batch, seq, d)
